# baseline (speedup 1.0000x reference)
import functools
import jax
import jax.numpy as jnp
from jax.experimental import pallas as pl
from jax.experimental.pallas import tpu as pltpu

_S = 64.0
_M = 0.4


def _msm_kernel(lab_ref, cos_ref, out_ref, acc_ref, vl_ref, *, bc, c_total, nblocks):
    pid = pl.program_id(0)

    @pl.when(pid == 0)
    def _init():
        acc_ref[...] = jnp.zeros_like(acc_ref)
        vl_ref[...] = jnp.zeros_like(vl_ref)

    x = cos_ref[...]
    b = x.shape[0]
    lab = lab_ref[...]
    colbase = pid * bc
    loc = lab - colbase
    ids = jax.lax.broadcasted_iota(jnp.int32, (b, bc), 1)
    is_lab = ids == loc
    v = x * _S
    vadj = jnp.where(is_lab, v - (_S * _M), v)
    e = jnp.exp(vadj)
    e = jnp.where(ids < (c_total - colbase), e, 0.0)
    vlc = jnp.where(is_lab, v, 0.0)

    acc = acc_ref[...]
    vl = vl_ref[...]
    for k in range(bc // 128):
        sl = slice(k * 128, (k + 1) * 128)
        acc = acc + e[:, sl]
        vl = vl + vlc[:, sl]
    acc_ref[...] = acc
    vl_ref[...] = vl

    @pl.when(pid == nblocks - 1)
    def _epilogue():
        accrow = jnp.sum(acc_ref[...], axis=1)
        vlrow = jnp.sum(vl_ref[...], axis=1)
        labv = lab_ref[...][:, 0]
        valid = labv != -1
        nll = jnp.log(accrow) - (vlrow - _S * _M)
        nll = jnp.where(valid, nll, 0.0)
        nv = jnp.maximum(jnp.sum(valid.astype(jnp.float32)), 1.0)
        out_ref[...] = (jnp.sum(nll) / nv).reshape(1, 1)


@jax.jit
def kernel(cosine, label):
    b, c = cosine.shape
    bc = 2048
    nblocks = pl.cdiv(c, bc)
    out = pl.pallas_call(
        functools.partial(_msm_kernel, bc=bc, c_total=c, nblocks=nblocks),
        grid=(nblocks,),
        in_specs=[
            pl.BlockSpec((b, 1), lambda i: (0, 0)),
            pl.BlockSpec((b, bc), lambda i: (0, i)),
        ],
        out_specs=pl.BlockSpec((1, 1), lambda i: (0, 0)),
        out_shape=jax.ShapeDtypeStruct((1, 1), jnp.float32),
        scratch_shapes=[
            pltpu.VMEM((b, 128), jnp.float32),
            pltpu.VMEM((b, 128), jnp.float32),
        ],
        compiler_params=pltpu.CompilerParams(
            dimension_semantics=("arbitrary",),
        ),
    )(label[:, None], cosine)
    return out[0, 0]
